# Initial kernel scaffold; baseline (speedup 1.0000x reference)
import jax
import jax.numpy as jnp
from jax.experimental import pallas as pl


def kernel(A_B_G_nonenormal_UV, A_B_G_nonenormal_VU, user_table, item_table, Wu, Wv):
    raise NotImplementedError("write your pallas kernel here")


# speedup vs baseline: 5.6007x; 5.6007x over previous
import jax
import jax.numpy as jnp
from jax.experimental import pallas as pl

_BM = 512


def _phase_a(vu_ref, user_ref, t1_ref):
    t1_ref[...] = jnp.dot(vu_ref[...], user_ref[...],
                          preferred_element_type=jnp.float32)


def _phase_b(uv_ref, item_ref, t1_ref, wu_ref, t2_ref, uh_ref):
    uv = uv_ref[...]
    item = item_ref[...]
    t2_ref[...] = jnp.dot(uv, item, preferred_element_type=jnp.float32)
    su = jnp.dot(uv, item + t1_ref[...], preferred_element_type=jnp.float32)
    uh_ref[...] = jax.nn.relu(
        jnp.dot(su, wu_ref[...], preferred_element_type=jnp.float32))


def _phase_c(vu_ref, user_ref, t2_ref, wv_ref, ih_ref):
    sv = jnp.dot(vu_ref[...], user_ref[...] + t2_ref[...],
                 preferred_element_type=jnp.float32)
    ih_ref[...] = jax.nn.relu(
        jnp.dot(sv, wv_ref[...], preferred_element_type=jnp.float32))


def kernel(A_B_G_nonenormal_UV, A_B_G_nonenormal_VU, user_table, item_table, Wu, Wv):
    UV, VU = A_B_G_nonenormal_UV, A_B_G_nonenormal_VU
    U, I = UV.shape
    D = user_table.shape[1]
    grid_u = U // _BM
    grid_i = I // _BM

    row_blk = lambda r, c: pl.BlockSpec((_BM, c), lambda i: (i, 0))
    full = lambda r, c: pl.BlockSpec((r, c), lambda i: (0, 0))

    t1 = pl.pallas_call(
        _phase_a,
        grid=(grid_i,),
        in_specs=[row_blk(I, U), full(U, D)],
        out_specs=row_blk(I, D),
        out_shape=jax.ShapeDtypeStruct((I, D), jnp.float32),
    )(VU, user_table)

    t2, user_h = pl.pallas_call(
        _phase_b,
        grid=(grid_u,),
        in_specs=[row_blk(U, I), full(I, D), full(I, D), full(D, D)],
        out_specs=[row_blk(U, D), row_blk(U, D)],
        out_shape=[jax.ShapeDtypeStruct((U, D), jnp.float32),
                   jax.ShapeDtypeStruct((U, D), jnp.float32)],
    )(UV, item_table, t1, Wu)

    item_h = pl.pallas_call(
        _phase_c,
        grid=(grid_i,),
        in_specs=[row_blk(I, U), full(U, D), full(U, D), full(D, D)],
        out_specs=row_blk(I, D),
        out_shape=jax.ShapeDtypeStruct((I, D), jnp.float32),
    )(VU, user_table, t2, Wv)

    return (user_h, item_h)
